# Initial kernel scaffold; baseline (speedup 1.0000x reference)
import jax
import jax.numpy as jnp
from jax.experimental import pallas as pl


def kernel(x_struct, x_esm, edge_index, edge_attr, pos, params):
    raise NotImplementedError("write your pallas kernel here")


# speedup vs baseline: 2.8155x; 2.8155x over previous
import functools

import jax
import jax.numpy as jnp
from jax import lax
from jax.experimental import pallas as pl
from jax.experimental.pallas import tpu as pltpu
from jax.experimental.pallas import tpu_sc as plsc

N = 10000
E = 320000
H = 128
DE = 16
COORD_SCALE = 0.1

NC = 2
NS = 16
NW = NC * NS
CH = 80

f32 = jnp.float32
i32 = jnp.int32


def _mesh():
    return plsc.VectorSubcoreMesh(core_axis_name="c", subcore_axis_name="s")


def _sc_gather(ts, td, pp, src, dst):
    per_w = E // NW
    n_ch = per_w // CH
    chunk_bytes = CH * H * 4

    @functools.partial(
        pl.kernel,
        out_type=(jax.ShapeDtypeStruct((E, H), f32),) * 4,
        mesh=_mesh(),
        scratch_types=(
            pltpu.VMEM((CH,), i32),
            pltpu.VMEM((CH,), i32),
            pltpu.VMEM((CH, H), f32),
            pltpu.VMEM((CH, H), f32),
            pltpu.VMEM((CH, H), f32),
            pltpu.VMEM((CH, H), f32),
            pltpu.SemaphoreType.DMA,
            pltpu.SemaphoreType.DMA,
        ),
    )
    def gather_k(ts_h, td_h, pp_h, src_h, dst_h,
                 gs_h, gd_h, ps_h, pd_h,
                 sidx, didx, bs, bd, bps, bpd, gsem, wsem):
        wid = lax.axis_index("s") * NC + lax.axis_index("c")
        base = wid * per_w

        def drain_wb(off):
            pltpu.make_async_copy(bs, gs_h.at[pl.ds(off, CH)], wsem).wait()
            pltpu.make_async_copy(bd, gd_h.at[pl.ds(off, CH)], wsem).wait()
            pltpu.make_async_copy(bps, ps_h.at[pl.ds(off, CH)], wsem).wait()
            pltpu.make_async_copy(bpd, pd_h.at[pl.ds(off, CH)], wsem).wait()

        def body(i, carry):
            off = base + i * CH

            @pl.when(i > 0)
            def _():
                drain_wb(off)

            pltpu.sync_copy(src_h.at[pl.ds(off, CH)], sidx)
            pltpu.sync_copy(dst_h.at[pl.ds(off, CH)], didx)
            c1 = pltpu.async_copy(ts_h.at[sidx], bs, gsem)
            c2 = pltpu.async_copy(td_h.at[didx], bd, gsem)
            c3 = pltpu.async_copy(pp_h.at[sidx], bps, gsem)
            c4 = pltpu.async_copy(pp_h.at[didx], bpd, gsem)
            c1.wait()
            c2.wait()
            c3.wait()
            c4.wait()
            pltpu.async_copy(bs, gs_h.at[pl.ds(off, CH)], wsem)
            pltpu.async_copy(bd, gd_h.at[pl.ds(off, CH)], wsem)
            pltpu.async_copy(bps, ps_h.at[pl.ds(off, CH)], wsem)
            pltpu.async_copy(bpd, pd_h.at[pl.ds(off, CH)], wsem)
            return carry

        lax.fori_loop(0, n_ch, body, 0)
        drain_wb(base)

    return gather_k(ts, td, pp, src, dst)


_RPT = 624
_RPT_LAST = N - 15 * _RPT


def _copy_tile_rows(s, src_ref, dst_ref):
    rbase = s * _RPT

    @pl.when(s < NS - 1)
    def _():
        pltpu.sync_copy(src_ref.at[pl.ds(rbase, _RPT)],
                        dst_ref.at[pl.ds(rbase, _RPT)])

    @pl.when(s == NS - 1)
    def _():
        pltpu.sync_copy(src_ref.at[pl.ds((NS - 1) * _RPT, _RPT_LAST)],
                        dst_ref.at[pl.ds((NS - 1) * _RPT, _RPT_LAST)])


def _sc_scatter2(m, tr, dst, zeros_n):
    per_t = E // NS
    n_pair = per_t // CH // 2
    chunk_bytes = CH * H * 4

    @functools.partial(
        pl.kernel,
        out_type=(jax.ShapeDtypeStruct((N, H), f32),) * 2,
        mesh=_mesh(),
        scratch_types=(
            pltpu.VMEM((CH,), i32),
            pltpu.VMEM((CH,), i32),
            pltpu.VMEM((CH, H), f32),
            pltpu.VMEM((CH, H), f32),
            pltpu.VMEM_SHARED((N, H), f32),
            pltpu.SemaphoreType.DMA,
        ),
    )
    def scatter_k(m_h, tr_h, dst_h, z_h, a_h, dp_h,
                  idxA, idxB, bufA, bufB, acc, asem):
        c = lax.axis_index("c")
        s = lax.axis_index("s")
        tbase = s * per_t

        _copy_tile_rows(s, z_h, acc)
        plsc.subcore_barrier()

        def drain_add():
            pltpu.make_async_copy(bufA, acc.at[idxA], asem).wait()
            pltpu.make_async_copy(bufB, acc.at[idxB], asem).wait()

        def run(src_arr):
            def body(j, carry):
                offA = tbase + (2 * j) * CH
                offB = tbase + (2 * j + 1) * CH

                @pl.when(j > 0)
                def _():
                    drain_add()

                pltpu.sync_copy(dst_h.at[pl.ds(offA, CH)], idxA)
                pltpu.sync_copy(src_arr.at[pl.ds(offA, CH)], bufA)
                pltpu.async_copy(bufA, acc.at[idxA], asem, add=True)
                pltpu.sync_copy(dst_h.at[pl.ds(offB, CH)], idxB)
                pltpu.sync_copy(src_arr.at[pl.ds(offB, CH)], bufB)
                pltpu.async_copy(bufB, acc.at[idxB], asem, add=True)
                return carry

            lax.fori_loop(0, n_pair, body, 0)
            drain_add()

        @pl.when(c == 0)
        def _():
            run(m_h)

        @pl.when(c == 1)
        def _():
            run(tr_h)

        plsc.subcore_barrier()

        @pl.when(c == 0)
        def _():
            _copy_tile_rows(s, acc, a_h)

        @pl.when(c == 1)
        def _():
            _copy_tile_rows(s, acc, dp_h)

    return scatter_k(m, tr, dst, zeros_n)


def _sc_scatter1(m, dst, zeros_n):
    per_c = E // NC
    per_t = per_c // NS
    n_ch = per_t // CH
    n_pair = n_ch // 2
    has_tail = n_ch % 2 == 1
    chunk_bytes = CH * H * 4

    @functools.partial(
        pl.kernel,
        out_type=(jax.ShapeDtypeStruct((N, H), f32),) * 2,
        mesh=_mesh(),
        scratch_types=(
            pltpu.VMEM((CH,), i32),
            pltpu.VMEM((CH,), i32),
            pltpu.VMEM((CH, H), f32),
            pltpu.VMEM((CH, H), f32),
            pltpu.VMEM_SHARED((N, H), f32),
            pltpu.SemaphoreType.DMA,
        ),
    )
    def scatter_k(m_h, dst_h, z_h, a0_h, a1_h,
                  idxA, idxB, bufA, bufB, acc, asem):
        c = lax.axis_index("c")
        s = lax.axis_index("s")
        tbase = c * per_c + s * per_t

        _copy_tile_rows(s, z_h, acc)
        plsc.subcore_barrier()

        def drain_add():
            pltpu.make_async_copy(bufA, acc.at[idxA], asem).wait()
            pltpu.make_async_copy(bufB, acc.at[idxB], asem).wait()

        def body(j, carry):
            offA = tbase + (2 * j) * CH
            offB = tbase + (2 * j + 1) * CH

            @pl.when(j > 0)
            def _():
                drain_add()

            pltpu.sync_copy(dst_h.at[pl.ds(offA, CH)], idxA)
            pltpu.sync_copy(m_h.at[pl.ds(offA, CH)], bufA)
            pltpu.async_copy(bufA, acc.at[idxA], asem, add=True)
            pltpu.sync_copy(dst_h.at[pl.ds(offB, CH)], idxB)
            pltpu.sync_copy(m_h.at[pl.ds(offB, CH)], bufB)
            pltpu.async_copy(bufB, acc.at[idxB], asem, add=True)
            return carry

        lax.fori_loop(0, n_pair, body, 0)
        drain_add()
        if has_tail:
            off = tbase + (n_ch - 1) * CH
            pltpu.sync_copy(dst_h.at[pl.ds(off, CH)], idxA)
            pltpu.sync_copy(m_h.at[pl.ds(off, CH)], bufA)
            pltpu.sync_copy(bufA, acc.at[idxA], add=True)

        plsc.subcore_barrier()

        @pl.when(c == 0)
        def _():
            _copy_tile_rows(s, acc, a0_h)

        @pl.when(c == 1)
        def _():
            _copy_tile_rows(s, acc, a1_h)

    return scatter_k(m, dst, zeros_n)


_BN = 1000
_BE = 4000


def _dot(a, b):
    return jnp.dot(a, b, preferred_element_type=f32)


def _full_spec(shape):
    return pl.BlockSpec(shape, lambda i: (0,) * len(shape))


def _row_spec(bs, width):
    return pl.BlockSpec((bs, width), lambda i: (i, 0))


def _fuse_body(xs_ref, xe_ref, ws, bs_, we, be_, wg1a, wg1b, bg1, wg2r, bg2r,
               ssr, ser, w1s, w1d, h_ref, ts_ref, td_ref):
    xs = xs_ref[...]
    xe = xe_ref[...]
    hs = _dot(xs, ws[...]) + bs_[...]
    he = _dot(xe, we[...]) + be_[...]
    g1 = jax.nn.relu(_dot(hs, wg1a[...]) + _dot(he, wg1b[...]) + bg1[...])
    gpre = jnp.sum(g1 * wg2r[...], axis=-1, keepdims=True) + bg2r[0:1, 0:1]
    gate = jax.nn.sigmoid(gpre)
    h = jax.nn.relu(gate * (ssr[...] * hs) + (1.0 - gate) * (ser[...] * he))
    h_ref[...] = h
    ts_ref[...] = _dot(h, w1s[...])
    td_ref[...] = _dot(h, w1d[...])


def _tc_fuse(xs, xe, p, w1s0, w1d0):
    wg1 = p["gate1"]["w"]
    args = (
        xs, xe,
        p["struct_proj"]["w"], p["struct_proj"]["b"].reshape(1, H),
        p["esm_proj"]["w"], p["esm_proj"]["b"].reshape(1, H),
        wg1[:H], wg1[H:], p["gate1"]["b"].reshape(1, H),
        p["gate2"]["w"].T, jnp.full((1, H), p["gate2"]["b"][0]),
        jnp.full((1, H), p["struct_scale"]), jnp.full((1, H), p["esm_scale"]),
        w1s0, w1d0,
    )
    in_specs = [
        _row_spec(_BN, H), _row_spec(_BN, 2 * H),
        _full_spec((H, H)), _full_spec((1, H)),
        _full_spec((2 * H, H)), _full_spec((1, H)),
        _full_spec((H, H)), _full_spec((H, H)), _full_spec((1, H)),
        _full_spec((1, H)), _full_spec((1, H)),
        _full_spec((1, H)), _full_spec((1, H)),
        _full_spec((H, H)), _full_spec((H, H)),
    ]
    return pl.pallas_call(
        _fuse_body,
        grid=(N // _BN,),
        in_specs=in_specs,
        out_specs=[_row_spec(_BN, H)] * 3,
        out_shape=[jax.ShapeDtypeStruct((N, H), f32)] * 3,
    )(*args)


def _edge_body_coord(gs_ref, gd_ref, ps_ref, pd_ref, ea_ref, w1dr, w1e, b1,
                     w2, b2, c1, bc1, c2r, bc2r, m_ref, tr_ref):
    diff = pd_ref[...] - ps_ref[...]
    d2 = jnp.clip(jnp.sum(diff * diff, axis=-1, keepdims=True), 0.0, 1000.0)
    pre = gs_ref[...] + gd_ref[...] + d2 * w1dr[...] \
        + _dot(ea_ref[...], w1e[...]) + b1[...]
    m = jax.nn.relu(_dot(jax.nn.relu(pre), w2[...]) + b2[...])
    m_ref[...] = m
    cc = jax.nn.relu(_dot(m, c1[...]) + bc1[...])
    coef = jnp.tanh(jnp.sum(cc * c2r[...], axis=-1, keepdims=True)
                    + bc2r[0:1, 0:1]) * COORD_SCALE
    tr_ref[...] = (diff / jnp.sqrt(d2 + 1e-8)) * coef


def _edge_body_nocoord(gs_ref, gd_ref, ps_ref, pd_ref, ea_ref, w1dr, w1e, b1,
                       w2, b2, m_ref):
    diff = pd_ref[...] - ps_ref[...]
    d2 = jnp.clip(jnp.sum(diff * diff, axis=-1, keepdims=True), 0.0, 1000.0)
    pre = gs_ref[...] + gd_ref[...] + d2 * w1dr[...] \
        + _dot(ea_ref[...], w1e[...]) + b1[...]
    m_ref[...] = jax.nn.relu(_dot(jax.nn.relu(pre), w2[...]) + b2[...])


def _tc_edge(gs, gd, psg, pdg, ea, lp, with_coord):
    w1 = lp["edge1"]["w"]
    w1dr = w1[2 * H:2 * H + 1]
    w1e = w1[2 * H + 1:]
    args = [gs, gd, psg, pdg, ea,
            w1dr, w1e, lp["edge1"]["b"].reshape(1, H),
            lp["edge2"]["w"], lp["edge2"]["b"].reshape(1, H)]
    in_specs = [_row_spec(_BE, H)] * 4 + [
        _row_spec(_BE, DE),
        _full_spec((1, H)), _full_spec((DE, H)), _full_spec((1, H)),
        _full_spec((H, H)), _full_spec((1, H)),
    ]
    if with_coord:
        args += [lp["coord1"]["w"], lp["coord1"]["b"].reshape(1, H),
                 lp["coord2"]["w"].T, jnp.full((1, H), lp["coord2"]["b"][0])]
        in_specs += [_full_spec((H, H)), _full_spec((1, H)),
                     _full_spec((1, H)), _full_spec((1, H))]
        return pl.pallas_call(
            _edge_body_coord,
            grid=(E // _BE,),
            in_specs=in_specs,
            out_specs=[_row_spec(_BE, H)] * 2,
            out_shape=[jax.ShapeDtypeStruct((E, H), f32)] * 2,
        )(*args)
    return pl.pallas_call(
        _edge_body_nocoord,
        grid=(E // _BE,),
        in_specs=in_specs,
        out_specs=_row_spec(_BE, H),
        out_shape=jax.ShapeDtypeStruct((E, H), f32),
    )(*args)


def _ln(x, g, b, eps=1e-5):
    mu = jnp.mean(x, axis=-1, keepdims=True)
    xc = x - mu
    var = jnp.mean(xc * xc, axis=-1, keepdims=True)
    return xc / jnp.sqrt(var + eps) * g + b


def _node_body(h_ref, a_ref, dp_ref, pp_ref, wna, wnb, bn, g, b, w1s, w1d,
               ho_ref, ts_ref, td_ref, ppo_ref):
    h = h_ref[...]
    hu = jax.nn.relu(_dot(h, wna[...]) + _dot(a_ref[...], wnb[...]) + bn[...])
    hn = _ln(h + hu, g[...], b[...])
    ho_ref[...] = hn
    ts_ref[...] = _dot(hn, w1s[...])
    td_ref[...] = _dot(hn, w1d[...])
    ppo_ref[...] = pp_ref[...] + dp_ref[...]


def _tc_node(h, a, dp, pp, lp, w1s_next, w1d_next):
    wn = lp["node1"]["w"]
    args = (h, a, dp, pp,
            wn[:H], wn[H:], lp["node1"]["b"].reshape(1, H),
            lp["node_norm"]["g"].reshape(1, H),
            lp["node_norm"]["b"].reshape(1, H),
            w1s_next, w1d_next)
    in_specs = [_row_spec(_BN, H)] * 4 + [
        _full_spec((H, H)), _full_spec((H, H)), _full_spec((1, H)),
        _full_spec((1, H)), _full_spec((1, H)),
        _full_spec((H, H)), _full_spec((H, H)),
    ]
    return pl.pallas_call(
        _node_body,
        grid=(N // _BN,),
        in_specs=in_specs,
        out_specs=[_row_spec(_BN, H)] * 4,
        out_shape=[jax.ShapeDtypeStruct((N, H), f32)] * 4,
    )(*args)


def _final_body(h_ref, a0_ref, a1_ref, wna, wnb, bn, g, b, gf, bf, wh, bh,
                out_ref):
    h = h_ref[...]
    agg = a0_ref[...] + a1_ref[...]
    hu = jax.nn.relu(_dot(h, wna[...]) + _dot(agg, wnb[...]) + bn[...])
    hn = _ln(h + hu, g[...], b[...])
    hf = _ln(hn, gf[...], bf[...])
    out_ref[...] = _dot(hf, wh[...]) + bh[...]


def _tc_final(h, a0, a1, lp, p):
    wn = lp["node1"]["w"]
    wh = jnp.pad(p["head"]["w"], ((0, 0), (0, H - 20)))
    bh = jnp.pad(p["head"]["b"], (0, H - 20)).reshape(1, H)
    args = (h, a0, a1,
            wn[:H], wn[H:], lp["node1"]["b"].reshape(1, H),
            lp["node_norm"]["g"].reshape(1, H),
            lp["node_norm"]["b"].reshape(1, H),
            p["final_norm"]["g"].reshape(1, H),
            p["final_norm"]["b"].reshape(1, H),
            wh, bh)
    in_specs = [_row_spec(_BN, H)] * 3 + [
        _full_spec((H, H)), _full_spec((H, H)), _full_spec((1, H)),
        _full_spec((1, H)), _full_spec((1, H)),
        _full_spec((1, H)), _full_spec((1, H)),
        _full_spec((H, H)), _full_spec((1, H)),
    ]
    return pl.pallas_call(
        _final_body,
        grid=(N // _BN,),
        in_specs=in_specs,
        out_specs=_row_spec(_BN, H),
        out_shape=jax.ShapeDtypeStruct((N, H), f32),
    )(*args)


def _w1_parts(lp):
    w1 = lp["edge1"]["w"]
    return w1[:H], w1[H:2 * H]


def kernel(x_struct, x_esm, edge_index, edge_attr, pos, params):
    src = edge_index[0]
    dst = edge_index[1]
    pp = jnp.pad(pos, ((0, 0), (0, H - 3)))
    zeros_n = jnp.zeros((N, H), f32)
    layers = params["layers"]

    w1s0, w1d0 = _w1_parts(layers[0])
    h, ts, td = _tc_fuse(x_struct, x_esm, params, w1s0, w1d0)

    for l in range(4):
        lp = layers[l]
        gs, gd, psg, pdg = _sc_gather(ts, td, pp, src, dst)
        if l < 3:
            m, tr = _tc_edge(gs, gd, psg, pdg, edge_attr, lp, True)
            a, dp = _sc_scatter2(m, tr, dst, zeros_n)
            w1s_n, w1d_n = _w1_parts(layers[l + 1])
            h, ts, td, pp = _tc_node(h, a, dp, pp, lp, w1s_n, w1d_n)
        else:
            m = _tc_edge(gs, gd, psg, pdg, edge_attr, lp, False)
            a0, a1 = _sc_scatter1(m, dst, zeros_n)
            out = _tc_final(h, a0, a1, lp, params)

    return out[:, :20]


# speedup vs baseline: 3.5690x; 1.2676x over previous
import functools

import jax
import jax.numpy as jnp
from jax import lax
from jax.experimental import pallas as pl
from jax.experimental.pallas import tpu as pltpu
from jax.experimental.pallas import tpu_sc as plsc

N = 10000
E = 320000
H = 128
DE = 16
PW = 16
COORD_SCALE = 0.1

NC = 2
NS = 16
NW = NC * NS
CH = 80

f32 = jnp.float32
i32 = jnp.int32


def _mesh():
    return plsc.VectorSubcoreMesh(core_axis_name="c", subcore_axis_name="s")


_RPT = 624
_RPT_LAST = N - (NS - 1) * _RPT


def _copy_tile_rows(s, src_ref, dst_ref):
    rbase = s * _RPT

    @pl.when(s < NS - 1)
    def _():
        pltpu.sync_copy(src_ref.at[pl.ds(rbase, _RPT)],
                        dst_ref.at[pl.ds(rbase, _RPT)])

    @pl.when(s == NS - 1)
    def _():
        pltpu.sync_copy(src_ref.at[pl.ds((NS - 1) * _RPT, _RPT_LAST)],
                        dst_ref.at[pl.ds((NS - 1) * _RPT, _RPT_LAST)])


def _sc_gather(ts, td, pp, src, dst):
    per_w = E // NW
    n_ch = per_w // CH
    n_pair = n_ch // 2
    has_tail = n_ch % 2 == 1

    @functools.partial(
        pl.kernel,
        out_type=(jax.ShapeDtypeStruct((E, H), f32),) * 4,
        mesh=_mesh(),
        scratch_types=(
            pltpu.VMEM((per_w,), i32),
            pltpu.VMEM((per_w,), i32),
            pltpu.VMEM((CH, H), f32), pltpu.VMEM((CH, H), f32),
            pltpu.VMEM((CH, H), f32), pltpu.VMEM((CH, H), f32),
            pltpu.VMEM((CH, H), f32), pltpu.VMEM((CH, H), f32),
            pltpu.VMEM((CH, H), f32), pltpu.VMEM((CH, H), f32),
            pltpu.SemaphoreType.DMA, pltpu.SemaphoreType.DMA,
            pltpu.SemaphoreType.DMA, pltpu.SemaphoreType.DMA,
        ),
    )
    def gather_k(ts_h, td_h, pp_h, src_h, dst_h,
                 gs_h, gd_h, ps_h, pd_h,
                 sall, dall, bsA, bsB, bdA, bdB, bpsA, bpsB, bpdA, bpdB,
                 gsemA, gsemB, wsemA, wsemB):
        wid = lax.axis_index("s") * NC + lax.axis_index("c")
        base = wid * per_w

        pltpu.sync_copy(src_h.at[pl.ds(base, per_w)], sall)
        pltpu.sync_copy(dst_h.at[pl.ds(base, per_w)], dall)

        def fire_gathers(loc, bs_, bd_, bps_, bpd_, sem):
            si = sall.at[pl.ds(loc, CH)]
            di = dall.at[pl.ds(loc, CH)]
            return (pltpu.async_copy(ts_h.at[si], bs_, sem),
                    pltpu.async_copy(td_h.at[di], bd_, sem),
                    pltpu.async_copy(pp_h.at[si], bps_, sem),
                    pltpu.async_copy(pp_h.at[di], bpd_, sem))

        def fire_wb(off, bs_, bd_, bps_, bpd_, sem):
            pltpu.async_copy(bs_, gs_h.at[pl.ds(off, CH)], sem)
            pltpu.async_copy(bd_, gd_h.at[pl.ds(off, CH)], sem)
            pltpu.async_copy(bps_, ps_h.at[pl.ds(off, CH)], sem)
            pltpu.async_copy(bpd_, pd_h.at[pl.ds(off, CH)], sem)

        def drain_wb(bs_, bd_, bps_, bpd_, sem):
            pltpu.make_async_copy(bs_, gs_h.at[pl.ds(base, CH)], sem).wait()
            pltpu.make_async_copy(bd_, gd_h.at[pl.ds(base, CH)], sem).wait()
            pltpu.make_async_copy(bps_, ps_h.at[pl.ds(base, CH)], sem).wait()
            pltpu.make_async_copy(bpd_, pd_h.at[pl.ds(base, CH)], sem).wait()

        def body(j, carry):
            locA = (2 * j) * CH
            locB = locA + CH

            @pl.when(j > 0)
            def _():
                drain_wb(bsA, bdA, bpsA, bpdA, wsemA)

            hA = fire_gathers(locA, bsA, bdA, bpsA, bpdA, gsemA)

            @pl.when(j > 0)
            def _():
                drain_wb(bsB, bdB, bpsB, bpdB, wsemB)

            hB = fire_gathers(locB, bsB, bdB, bpsB, bpdB, gsemB)
            for h in hA:
                h.wait()
            fire_wb(base + locA, bsA, bdA, bpsA, bpdA, wsemA)
            for h in hB:
                h.wait()
            fire_wb(base + locB, bsB, bdB, bpsB, bpdB, wsemB)
            return carry

        lax.fori_loop(0, n_pair, body, 0)
        if has_tail:
            loc = (n_ch - 1) * CH
            drain_wb(bsA, bdA, bpsA, bpdA, wsemA)
            hA = fire_gathers(loc, bsA, bdA, bpsA, bpdA, gsemA)
            for h in hA:
                h.wait()
            fire_wb(base + loc, bsA, bdA, bpsA, bpdA, wsemA)
        drain_wb(bsA, bdA, bpsA, bpdA, wsemA)
        drain_wb(bsB, bdB, bpsB, bpdB, wsemB)

    return gather_k(ts, td, pp, src, dst)


def _sc_scatter2(m, tr, dst, zeros_n):
    per_t = E // NS
    n_pair = per_t // CH // 2

    @functools.partial(
        pl.kernel,
        out_type=(jax.ShapeDtypeStruct((N, H), f32),) * 2,
        mesh=_mesh(),
        scratch_types=(
            pltpu.VMEM((CH,), i32), pltpu.VMEM((CH,), i32),
            pltpu.VMEM((CH, H), f32), pltpu.VMEM((CH, H), f32),
            pltpu.VMEM((CH, H), f32), pltpu.VMEM((CH, H), f32),
            pltpu.VMEM_SHARED((N, H), f32),
            pltpu.SemaphoreType.DMA, pltpu.SemaphoreType.DMA,
        ),
    )
    def scatter_k(m_h, tr_h, dst_h, z_h, a_h, dp_h,
                  idxA, idxB, mA, mB, tA, tB, acc, asemA, asemB):
        c = lax.axis_index("c")
        s = lax.axis_index("s")
        tbase = s * per_t

        _copy_tile_rows(s, z_h, acc)
        plsc.subcore_barrier()

        def run(src_arr, accum, bufA, bufB):
            def drain_add(buf, idx, sem):
                pltpu.make_async_copy(buf, accum.at[idx], sem).wait()

            def body(j, carry):
                offA = tbase + (2 * j) * CH
                offB = offA + CH

                @pl.when(j > 0)
                def _():
                    drain_add(bufA, idxA, asemA)

                hiA = pltpu.async_copy(dst_h.at[pl.ds(offA, CH)], idxA, asemA)
                hrA = pltpu.async_copy(src_arr.at[pl.ds(offA, CH)], bufA, asemA)

                @pl.when(j > 0)
                def _():
                    drain_add(bufB, idxB, asemB)

                hiB = pltpu.async_copy(dst_h.at[pl.ds(offB, CH)], idxB, asemB)
                hrB = pltpu.async_copy(src_arr.at[pl.ds(offB, CH)], bufB, asemB)
                hiA.wait()
                hrA.wait()
                pltpu.async_copy(bufA, accum.at[idxA], asemA, add=True)
                hiB.wait()
                hrB.wait()
                pltpu.async_copy(bufB, accum.at[idxB], asemB, add=True)
                return carry

            lax.fori_loop(0, n_pair, body, 0)
            drain_add(bufA, idxA, asemA)
            drain_add(bufB, idxB, asemB)

        @pl.when(c == 0)
        def _():
            run(m_h, acc, mA, mB)

        @pl.when(c == 1)
        def _():
            run(tr_h, acc, tA, tB)

        plsc.subcore_barrier()

        @pl.when(c == 0)
        def _():
            _copy_tile_rows(s, acc, a_h)

        @pl.when(c == 1)
        def _():
            _copy_tile_rows(s, acc, dp_h)

    return scatter_k(m, tr, dst, zeros_n)


def _sc_scatter1(m, dst, zeros_n):
    per_c = E // NC
    per_t = per_c // NS
    n_ch = per_t // CH
    n_pair = n_ch // 2
    has_tail = n_ch % 2 == 1

    @functools.partial(
        pl.kernel,
        out_type=(jax.ShapeDtypeStruct((N, H), f32),) * 2,
        mesh=_mesh(),
        scratch_types=(
            pltpu.VMEM((CH,), i32), pltpu.VMEM((CH,), i32),
            pltpu.VMEM((CH, H), f32), pltpu.VMEM((CH, H), f32),
            pltpu.VMEM_SHARED((N, H), f32),
            pltpu.SemaphoreType.DMA, pltpu.SemaphoreType.DMA,
        ),
    )
    def scatter_k(m_h, dst_h, z_h, a0_h, a1_h,
                  idxA, idxB, bufA, bufB, acc, asemA, asemB):
        c = lax.axis_index("c")
        s = lax.axis_index("s")
        tbase = c * per_c + s * per_t

        _copy_tile_rows(s, z_h, acc)
        plsc.subcore_barrier()

        def drain_add(buf, idx, sem):
            pltpu.make_async_copy(buf, acc.at[idx], sem).wait()

        def chunk(off, idx, buf, sem, first):
            @pl.when(jnp.logical_not(first))
            def _():
                drain_add(buf, idx, sem)

            hi = pltpu.async_copy(dst_h.at[pl.ds(off, CH)], idx, sem)
            hr = pltpu.async_copy(m_h.at[pl.ds(off, CH)], buf, sem)
            return hi, hr

        def body(j, carry):
            offA = tbase + (2 * j) * CH
            offB = offA + CH
            hiA, hrA = chunk(offA, idxA, bufA, asemA, j == 0)
            hiB, hrB = chunk(offB, idxB, bufB, asemB, j == 0)
            hiA.wait()
            hrA.wait()
            pltpu.async_copy(bufA, acc.at[idxA], asemA, add=True)
            hiB.wait()
            hrB.wait()
            pltpu.async_copy(bufB, acc.at[idxB], asemB, add=True)
            return carry

        lax.fori_loop(0, n_pair, body, 0)
        if has_tail:
            off = tbase + (n_ch - 1) * CH
            drain_add(bufA, idxA, asemA)
            hi = pltpu.async_copy(dst_h.at[pl.ds(off, CH)], idxA, asemA)
            hr = pltpu.async_copy(m_h.at[pl.ds(off, CH)], bufA, asemA)
            hi.wait()
            hr.wait()
            pltpu.async_copy(bufA, acc.at[idxA], asemA, add=True)
        drain_add(bufA, idxA, asemA)
        drain_add(bufB, idxB, asemB)

        plsc.subcore_barrier()

        @pl.when(c == 0)
        def _():
            _copy_tile_rows(s, acc, a0_h)

        @pl.when(c == 1)
        def _():
            _copy_tile_rows(s, acc, a1_h)

    return scatter_k(m, dst, zeros_n)


_BN = 1000
_BE = 4000


def _dot(a, b):
    return jnp.dot(a, b, preferred_element_type=f32)


def _full_spec(shape):
    return pl.BlockSpec(shape, lambda i: (0,) * len(shape))


def _row_spec(bs, width):
    return pl.BlockSpec((bs, width), lambda i: (i, 0))


def _fuse_body(xs_ref, xe_ref, ws, bs_, we, be_, wg1a, wg1b, bg1, wg2r, bg2r,
               ssr, ser, w1s, w1d, h_ref, ts_ref, td_ref):
    xs = xs_ref[...]
    xe = xe_ref[...]
    hs = _dot(xs, ws[...]) + bs_[...]
    he = _dot(xe, we[...]) + be_[...]
    g1 = jax.nn.relu(_dot(hs, wg1a[...]) + _dot(he, wg1b[...]) + bg1[...])
    gpre = jnp.sum(g1 * wg2r[...], axis=-1, keepdims=True) + bg2r[0:1, 0:1]
    gate = jax.nn.sigmoid(gpre)
    h = jax.nn.relu(gate * (ssr[...] * hs) + (1.0 - gate) * (ser[...] * he))
    h_ref[...] = h
    ts_ref[...] = _dot(h, w1s[...])
    td_ref[...] = _dot(h, w1d[...])


def _tc_fuse(xs, xe, p, w1s0, w1d0):
    wg1 = p["gate1"]["w"]
    args = (
        xs, xe,
        p["struct_proj"]["w"], p["struct_proj"]["b"].reshape(1, H),
        p["esm_proj"]["w"], p["esm_proj"]["b"].reshape(1, H),
        wg1[:H], wg1[H:], p["gate1"]["b"].reshape(1, H),
        p["gate2"]["w"].T, jnp.full((1, H), p["gate2"]["b"][0]),
        jnp.full((1, H), p["struct_scale"]), jnp.full((1, H), p["esm_scale"]),
        w1s0, w1d0,
    )
    in_specs = [
        _row_spec(_BN, H), _row_spec(_BN, 2 * H),
        _full_spec((H, H)), _full_spec((1, H)),
        _full_spec((2 * H, H)), _full_spec((1, H)),
        _full_spec((H, H)), _full_spec((H, H)), _full_spec((1, H)),
        _full_spec((1, H)), _full_spec((1, H)),
        _full_spec((1, H)), _full_spec((1, H)),
        _full_spec((H, H)), _full_spec((H, H)),
    ]
    return pl.pallas_call(
        _fuse_body,
        grid=(N // _BN,),
        in_specs=in_specs,
        out_specs=[_row_spec(_BN, H)] * 3,
        out_shape=[jax.ShapeDtypeStruct((N, H), f32)] * 3,
    )(*args)


def _edge_body_coord(gs_ref, gd_ref, ps_ref, pd_ref, ea_ref, w1dr, w1e, b1,
                     w2, b2, c1, bc1, c2r, bc2r, m_ref, tr_ref):
    diff = pd_ref[...] - ps_ref[...]
    d2 = jnp.clip(jnp.sum(diff * diff, axis=-1, keepdims=True), 0.0, 1000.0)
    pre = gs_ref[...] + gd_ref[...] + d2 * w1dr[...] \
        + _dot(ea_ref[...], w1e[...]) + b1[...]
    m = jax.nn.relu(_dot(jax.nn.relu(pre), w2[...]) + b2[...])
    m_ref[...] = m
    cc = jax.nn.relu(_dot(m, c1[...]) + bc1[...])
    coef = jnp.tanh(jnp.sum(cc * c2r[...], axis=-1, keepdims=True)
                    + bc2r[0:1, 0:1]) * COORD_SCALE
    tr_ref[...] = (diff / jnp.sqrt(d2 + 1e-8)) * coef


def _edge_body_nocoord(gs_ref, gd_ref, ps_ref, pd_ref, ea_ref, w1dr, w1e, b1,
                       w2, b2, m_ref):
    diff = pd_ref[...] - ps_ref[...]
    d2 = jnp.clip(jnp.sum(diff * diff, axis=-1, keepdims=True), 0.0, 1000.0)
    pre = gs_ref[...] + gd_ref[...] + d2 * w1dr[...] \
        + _dot(ea_ref[...], w1e[...]) + b1[...]
    m_ref[...] = jax.nn.relu(_dot(jax.nn.relu(pre), w2[...]) + b2[...])


def _tc_edge(gs, gd, psg, pdg, ea, lp, with_coord):
    w1 = lp["edge1"]["w"]
    w1dr = w1[2 * H:2 * H + 1]
    w1e = w1[2 * H + 1:]
    args = [gs, gd, psg, pdg, ea,
            w1dr, w1e, lp["edge1"]["b"].reshape(1, H),
            lp["edge2"]["w"], lp["edge2"]["b"].reshape(1, H)]
    in_specs = [_row_spec(_BE, H), _row_spec(_BE, H),
                _row_spec(_BE, H), _row_spec(_BE, H),
                _row_spec(_BE, DE),
                _full_spec((1, H)), _full_spec((DE, H)), _full_spec((1, H)),
                _full_spec((H, H)), _full_spec((1, H))]
    if with_coord:
        args += [lp["coord1"]["w"], lp["coord1"]["b"].reshape(1, H),
                 lp["coord2"]["w"].T, jnp.full((1, H), lp["coord2"]["b"][0])]
        in_specs += [_full_spec((H, H)), _full_spec((1, H)),
                     _full_spec((1, H)), _full_spec((1, H))]
        return pl.pallas_call(
            _edge_body_coord,
            grid=(E // _BE,),
            in_specs=in_specs,
            out_specs=[_row_spec(_BE, H)] * 2,
            out_shape=[jax.ShapeDtypeStruct((E, H), f32)] * 2,
        )(*args)
    return pl.pallas_call(
        _edge_body_nocoord,
        grid=(E // _BE,),
        in_specs=in_specs,
        out_specs=_row_spec(_BE, H),
        out_shape=jax.ShapeDtypeStruct((E, H), f32),
    )(*args)


def _ln(x, g, b, eps=1e-5):
    mu = jnp.mean(x, axis=-1, keepdims=True)
    xc = x - mu
    var = jnp.mean(xc * xc, axis=-1, keepdims=True)
    return xc / jnp.sqrt(var + eps) * g + b


def _node_body(h_ref, a_ref, dp_ref, pp_ref, wna, wnb, bn, g, b, w1s, w1d,
               ho_ref, ts_ref, td_ref, ppo_ref):
    h = h_ref[...]
    hu = jax.nn.relu(_dot(h, wna[...]) + _dot(a_ref[...], wnb[...]) + bn[...])
    hn = _ln(h + hu, g[...], b[...])
    ho_ref[...] = hn
    ts_ref[...] = _dot(hn, w1s[...])
    td_ref[...] = _dot(hn, w1d[...])
    ppo_ref[...] = pp_ref[...] + dp_ref[...]


def _tc_node(h, a, dp, pp, lp, w1s_next, w1d_next):
    wn = lp["node1"]["w"]
    args = (h, a, dp, pp,
            wn[:H], wn[H:], lp["node1"]["b"].reshape(1, H),
            lp["node_norm"]["g"].reshape(1, H),
            lp["node_norm"]["b"].reshape(1, H),
            w1s_next, w1d_next)
    in_specs = [_row_spec(_BN, H)] * 4 + [
        _full_spec((H, H)), _full_spec((H, H)), _full_spec((1, H)),
        _full_spec((1, H)), _full_spec((1, H)),
        _full_spec((H, H)), _full_spec((H, H))]
    return pl.pallas_call(
        _node_body,
        grid=(N // _BN,),
        in_specs=in_specs,
        out_specs=[_row_spec(_BN, H)] * 4,
        out_shape=[jax.ShapeDtypeStruct((N, H), f32)] * 4,
    )(*args)


def _final_body(h_ref, a0_ref, a1_ref, wna, wnb, bn, g, b, gf, bf, wh, bh,
                out_ref):
    h = h_ref[...]
    agg = a0_ref[...] + a1_ref[...]
    hu = jax.nn.relu(_dot(h, wna[...]) + _dot(agg, wnb[...]) + bn[...])
    hn = _ln(h + hu, g[...], b[...])
    hf = _ln(hn, gf[...], bf[...])
    out_ref[...] = _dot(hf, wh[...]) + bh[...]


def _tc_final(h, a0, a1, lp, p):
    wn = lp["node1"]["w"]
    wh = jnp.pad(p["head"]["w"], ((0, 0), (0, H - 20)))
    bh = jnp.pad(p["head"]["b"], (0, H - 20)).reshape(1, H)
    args = (h, a0, a1,
            wn[:H], wn[H:], lp["node1"]["b"].reshape(1, H),
            lp["node_norm"]["g"].reshape(1, H),
            lp["node_norm"]["b"].reshape(1, H),
            p["final_norm"]["g"].reshape(1, H),
            p["final_norm"]["b"].reshape(1, H),
            wh, bh)
    in_specs = [_row_spec(_BN, H)] * 3 + [
        _full_spec((H, H)), _full_spec((H, H)), _full_spec((1, H)),
        _full_spec((1, H)), _full_spec((1, H)),
        _full_spec((1, H)), _full_spec((1, H)),
        _full_spec((H, H)), _full_spec((1, H)),
    ]
    return pl.pallas_call(
        _final_body,
        grid=(N // _BN,),
        in_specs=in_specs,
        out_specs=_row_spec(_BN, H),
        out_shape=jax.ShapeDtypeStruct((N, H), f32),
    )(*args)


def _w1_parts(lp):
    w1 = lp["edge1"]["w"]
    return w1[:H], w1[H:2 * H]


def kernel(x_struct, x_esm, edge_index, edge_attr, pos, params):
    src = edge_index[0]
    dst = edge_index[1]
    pp = jnp.pad(pos, ((0, 0), (0, H - 3)))
    zeros_n = jnp.zeros((N, H), f32)
    layers = params["layers"]

    w1s0, w1d0 = _w1_parts(layers[0])
    h, ts, td = _tc_fuse(x_struct, x_esm, params, w1s0, w1d0)

    for l in range(4):
        lp = layers[l]
        gs, gd, psg, pdg = _sc_gather(ts, td, pp, src, dst)
        if l < 3:
            m, tr = _tc_edge(gs, gd, psg, pdg, edge_attr, lp, True)
            a, dp = _sc_scatter2(m, tr, dst, zeros_n)
            w1s_n, w1d_n = _w1_parts(layers[l + 1])
            h, ts, td, pp = _tc_node(h, a, dp, pp, lp, w1s_n, w1d_n)
        else:
            m = _tc_edge(gs, gd, psg, pdg, edge_attr, lp, False)
            a0, a1 = _sc_scatter1(m, dst, zeros_n)
            out = _tc_final(h, a0, a1, lp, params)

    return out[:, :20]


# speedup vs baseline: 4.2130x; 1.1805x over previous
import functools

import jax
import jax.numpy as jnp
from jax import lax
from jax.experimental import pallas as pl
from jax.experimental.pallas import tpu as pltpu
from jax.experimental.pallas import tpu_sc as plsc

N = 10000
E = 320000
H = 128
DE = 16
PW = 16
COORD_SCALE = 0.1

NC = 2
NS = 16
NW = NC * NS
CH = 80

f32 = jnp.float32
i32 = jnp.int32


def _mesh():
    return plsc.VectorSubcoreMesh(core_axis_name="c", subcore_axis_name="s")


_RPT = 624
_RPT_LAST = N - (NS - 1) * _RPT


def _copy_tile_rows(s, src_ref, dst_ref):
    rbase = s * _RPT

    @pl.when(s < NS - 1)
    def _():
        pltpu.sync_copy(src_ref.at[pl.ds(rbase, _RPT)],
                        dst_ref.at[pl.ds(rbase, _RPT)])

    @pl.when(s == NS - 1)
    def _():
        pltpu.sync_copy(src_ref.at[pl.ds((NS - 1) * _RPT, _RPT_LAST)],
                        dst_ref.at[pl.ds((NS - 1) * _RPT, _RPT_LAST)])


def _sc_gather(ts, td, pp, pn, src, dst):
    per_w = E // NW
    n_ch = per_w // CH
    n_pair = n_ch // 2
    has_tail = n_ch % 2 == 1

    @functools.partial(
        pl.kernel,
        out_type=(jax.ShapeDtypeStruct((E, H), f32),) * 2,
        mesh=_mesh(),
        scratch_types=(
            pltpu.VMEM((per_w,), i32),
            pltpu.VMEM((per_w,), i32),
            pltpu.VMEM((CH, H), f32), pltpu.VMEM((CH, H), f32),
            pltpu.VMEM((CH, H), f32), pltpu.VMEM((CH, H), f32),
            pltpu.SemaphoreType.DMA, pltpu.SemaphoreType.DMA,
            pltpu.SemaphoreType.DMA, pltpu.SemaphoreType.DMA,
        ),
    )
    def gather_k(ts_h, td_h, pp_h, pn_h, src_h, dst_h,
                 gsum_h, pdif_h,
                 sall, dall, bgA, bgB, bpA, bpB,
                 gsemA, gsemB, wsemA, wsemB):
        wid = lax.axis_index("s") * NC + lax.axis_index("c")
        base = wid * per_w

        pltpu.sync_copy(src_h.at[pl.ds(base, per_w)], sall)
        pltpu.sync_copy(dst_h.at[pl.ds(base, per_w)], dall)

        def fire_phase1(loc, bg_, bp_, sem):
            si = sall.at[pl.ds(loc, CH)]
            di = dall.at[pl.ds(loc, CH)]
            return (pltpu.async_copy(ts_h.at[si], bg_, sem),
                    pltpu.async_copy(pp_h.at[di], bp_, sem))

        def fire_phase2(loc, bg_, bp_, sem):
            si = sall.at[pl.ds(loc, CH)]
            di = dall.at[pl.ds(loc, CH)]
            return (pltpu.async_copy(td_h.at[di], bg_, sem, add=True),
                    pltpu.async_copy(pn_h.at[si], bp_, sem, add=True))

        def fire_wb(off, bg_, bp_, sem):
            pltpu.async_copy(bg_, gsum_h.at[pl.ds(off, CH)], sem)
            pltpu.async_copy(bp_, pdif_h.at[pl.ds(off, CH)], sem)

        def drain_wb(bg_, bp_, sem):
            pltpu.make_async_copy(bg_, gsum_h.at[pl.ds(base, CH)], sem).wait()
            pltpu.make_async_copy(bp_, pdif_h.at[pl.ds(base, CH)], sem).wait()

        def run_chunk_pair(jA_loc, jB_loc, first):
            @pl.when(jnp.logical_not(first))
            def _():
                drain_wb(bgA, bpA, wsemA)

            h1A = fire_phase1(jA_loc, bgA, bpA, gsemA)

            @pl.when(jnp.logical_not(first))
            def _():
                drain_wb(bgB, bpB, wsemB)

            h1B = fire_phase1(jB_loc, bgB, bpB, gsemB)
            for h in h1A:
                h.wait()
            h2A = fire_phase2(jA_loc, bgA, bpA, gsemA)
            for h in h1B:
                h.wait()
            h2B = fire_phase2(jB_loc, bgB, bpB, gsemB)
            for h in h2A:
                h.wait()
            fire_wb(base + jA_loc, bgA, bpA, wsemA)
            for h in h2B:
                h.wait()
            fire_wb(base + jB_loc, bgB, bpB, wsemB)

        def body(j, carry):
            locA = (2 * j) * CH
            run_chunk_pair(locA, locA + CH, j == 0)
            return carry

        lax.fori_loop(0, n_pair, body, 0)
        if has_tail:
            loc = (n_ch - 1) * CH
            drain_wb(bgA, bpA, wsemA)
            h1 = fire_phase1(loc, bgA, bpA, gsemA)
            for h in h1:
                h.wait()
            h2 = fire_phase2(loc, bgA, bpA, gsemA)
            for h in h2:
                h.wait()
            fire_wb(base + loc, bgA, bpA, wsemA)
        drain_wb(bgA, bpA, wsemA)
        drain_wb(bgB, bpB, wsemB)

    return gather_k(ts, td, pp, pn, src, dst)


def _sc_scatter2(m, tr, dst, zeros_n):
    per_t = E // NS
    n_pair = per_t // CH // 2

    @functools.partial(
        pl.kernel,
        out_type=(jax.ShapeDtypeStruct((N, H), f32),) * 2,
        mesh=_mesh(),
        scratch_types=(
            pltpu.VMEM((CH,), i32), pltpu.VMEM((CH,), i32),
            pltpu.VMEM((CH, H), f32), pltpu.VMEM((CH, H), f32),
            pltpu.VMEM((CH, H), f32), pltpu.VMEM((CH, H), f32),
            pltpu.VMEM_SHARED((N, H), f32),
            pltpu.SemaphoreType.DMA, pltpu.SemaphoreType.DMA,
        ),
    )
    def scatter_k(m_h, tr_h, dst_h, z_h, a_h, dp_h,
                  idxA, idxB, mA, mB, tA, tB, acc, asemA, asemB):
        c = lax.axis_index("c")
        s = lax.axis_index("s")
        tbase = s * per_t

        _copy_tile_rows(s, z_h, acc)
        plsc.subcore_barrier()

        def run(src_arr, accum, bufA, bufB):
            def drain_add(buf, idx, sem):
                pltpu.make_async_copy(buf, accum.at[idx], sem).wait()

            def body(j, carry):
                offA = tbase + (2 * j) * CH
                offB = offA + CH

                @pl.when(j > 0)
                def _():
                    drain_add(bufA, idxA, asemA)

                hiA = pltpu.async_copy(dst_h.at[pl.ds(offA, CH)], idxA, asemA)
                hrA = pltpu.async_copy(src_arr.at[pl.ds(offA, CH)], bufA, asemA)

                @pl.when(j > 0)
                def _():
                    drain_add(bufB, idxB, asemB)

                hiB = pltpu.async_copy(dst_h.at[pl.ds(offB, CH)], idxB, asemB)
                hrB = pltpu.async_copy(src_arr.at[pl.ds(offB, CH)], bufB, asemB)
                hiA.wait()
                hrA.wait()
                pltpu.async_copy(bufA, accum.at[idxA], asemA, add=True)
                hiB.wait()
                hrB.wait()
                pltpu.async_copy(bufB, accum.at[idxB], asemB, add=True)
                return carry

            lax.fori_loop(0, n_pair, body, 0)
            drain_add(bufA, idxA, asemA)
            drain_add(bufB, idxB, asemB)

        @pl.when(c == 0)
        def _():
            run(m_h, acc, mA, mB)

        @pl.when(c == 1)
        def _():
            run(tr_h, acc, tA, tB)

        plsc.subcore_barrier()

        @pl.when(c == 0)
        def _():
            _copy_tile_rows(s, acc, a_h)

        @pl.when(c == 1)
        def _():
            _copy_tile_rows(s, acc, dp_h)

    return scatter_k(m, tr, dst, zeros_n)


def _sc_scatter1(m, dst, zeros_n):
    per_c = E // NC
    per_t = per_c // NS
    n_ch = per_t // CH
    n_pair = n_ch // 2
    has_tail = n_ch % 2 == 1

    @functools.partial(
        pl.kernel,
        out_type=(jax.ShapeDtypeStruct((N, H), f32),) * 2,
        mesh=_mesh(),
        scratch_types=(
            pltpu.VMEM((CH,), i32), pltpu.VMEM((CH,), i32),
            pltpu.VMEM((CH, H), f32), pltpu.VMEM((CH, H), f32),
            pltpu.VMEM_SHARED((N, H), f32),
            pltpu.SemaphoreType.DMA, pltpu.SemaphoreType.DMA,
        ),
    )
    def scatter_k(m_h, dst_h, z_h, a0_h, a1_h,
                  idxA, idxB, bufA, bufB, acc, asemA, asemB):
        c = lax.axis_index("c")
        s = lax.axis_index("s")
        tbase = c * per_c + s * per_t

        _copy_tile_rows(s, z_h, acc)
        plsc.subcore_barrier()

        def drain_add(buf, idx, sem):
            pltpu.make_async_copy(buf, acc.at[idx], sem).wait()

        def chunk(off, idx, buf, sem, first):
            @pl.when(jnp.logical_not(first))
            def _():
                drain_add(buf, idx, sem)

            hi = pltpu.async_copy(dst_h.at[pl.ds(off, CH)], idx, sem)
            hr = pltpu.async_copy(m_h.at[pl.ds(off, CH)], buf, sem)
            return hi, hr

        def body(j, carry):
            offA = tbase + (2 * j) * CH
            offB = offA + CH
            hiA, hrA = chunk(offA, idxA, bufA, asemA, j == 0)
            hiB, hrB = chunk(offB, idxB, bufB, asemB, j == 0)
            hiA.wait()
            hrA.wait()
            pltpu.async_copy(bufA, acc.at[idxA], asemA, add=True)
            hiB.wait()
            hrB.wait()
            pltpu.async_copy(bufB, acc.at[idxB], asemB, add=True)
            return carry

        lax.fori_loop(0, n_pair, body, 0)
        if has_tail:
            off = tbase + (n_ch - 1) * CH
            drain_add(bufA, idxA, asemA)
            hi = pltpu.async_copy(dst_h.at[pl.ds(off, CH)], idxA, asemA)
            hr = pltpu.async_copy(m_h.at[pl.ds(off, CH)], bufA, asemA)
            hi.wait()
            hr.wait()
            pltpu.async_copy(bufA, acc.at[idxA], asemA, add=True)
        drain_add(bufA, idxA, asemA)
        drain_add(bufB, idxB, asemB)

        plsc.subcore_barrier()

        @pl.when(c == 0)
        def _():
            _copy_tile_rows(s, acc, a0_h)

        @pl.when(c == 1)
        def _():
            _copy_tile_rows(s, acc, a1_h)

    return scatter_k(m, dst, zeros_n)


_BN = 1000
_BE = 4000


def _dot(a, b):
    return jnp.dot(a, b, preferred_element_type=f32)


def _full_spec(shape):
    return pl.BlockSpec(shape, lambda i: (0,) * len(shape))


def _row_spec(bs, width):
    return pl.BlockSpec((bs, width), lambda i: (i, 0))


def _fuse_body(xs_ref, xe_ref, ws, bs_, we, be_, wg1a, wg1b, bg1, wg2r, bg2r,
               ssr, ser, w1s, w1d, h_ref, ts_ref, td_ref):
    xs = xs_ref[...]
    xe = xe_ref[...]
    hs = _dot(xs, ws[...]) + bs_[...]
    he = _dot(xe, we[...]) + be_[...]
    g1 = jax.nn.relu(_dot(hs, wg1a[...]) + _dot(he, wg1b[...]) + bg1[...])
    gpre = jnp.sum(g1 * wg2r[...], axis=-1, keepdims=True) + bg2r[0:1, 0:1]
    gate = jax.nn.sigmoid(gpre)
    h = jax.nn.relu(gate * (ssr[...] * hs) + (1.0 - gate) * (ser[...] * he))
    h_ref[...] = h
    ts_ref[...] = _dot(h, w1s[...])
    td_ref[...] = _dot(h, w1d[...])


def _tc_fuse(xs, xe, p, w1s0, w1d0):
    wg1 = p["gate1"]["w"]
    args = (
        xs, xe,
        p["struct_proj"]["w"], p["struct_proj"]["b"].reshape(1, H),
        p["esm_proj"]["w"], p["esm_proj"]["b"].reshape(1, H),
        wg1[:H], wg1[H:], p["gate1"]["b"].reshape(1, H),
        p["gate2"]["w"].T, jnp.full((1, H), p["gate2"]["b"][0]),
        jnp.full((1, H), p["struct_scale"]), jnp.full((1, H), p["esm_scale"]),
        w1s0, w1d0,
    )
    in_specs = [
        _row_spec(_BN, H), _row_spec(_BN, 2 * H),
        _full_spec((H, H)), _full_spec((1, H)),
        _full_spec((2 * H, H)), _full_spec((1, H)),
        _full_spec((H, H)), _full_spec((H, H)), _full_spec((1, H)),
        _full_spec((1, H)), _full_spec((1, H)),
        _full_spec((1, H)), _full_spec((1, H)),
        _full_spec((H, H)), _full_spec((H, H)),
    ]
    return pl.pallas_call(
        _fuse_body,
        grid=(N // _BN,),
        in_specs=in_specs,
        out_specs=[_row_spec(_BN, H)] * 3,
        out_shape=[jax.ShapeDtypeStruct((N, H), f32)] * 3,
    )(*args)


def _edge_body_coord(gsum_ref, pdif_ref, ea_ref, w1dr, w1e, b1,
                     w2, b2, c1, bc1, c2r, bc2r, m_ref, tr_ref):
    diff = pdif_ref[...]
    d2 = jnp.clip(jnp.sum(diff * diff, axis=-1, keepdims=True), 0.0, 1000.0)
    pre = gsum_ref[...] + d2 * w1dr[...] \
        + _dot(ea_ref[...], w1e[...]) + b1[...]
    m = jax.nn.relu(_dot(jax.nn.relu(pre), w2[...]) + b2[...])
    m_ref[...] = m
    cc = jax.nn.relu(_dot(m, c1[...]) + bc1[...])
    coef = jnp.tanh(jnp.sum(cc * c2r[...], axis=-1, keepdims=True)
                    + bc2r[0:1, 0:1]) * COORD_SCALE
    tr_ref[...] = (diff / jnp.sqrt(d2 + 1e-8)) * coef


def _edge_body_nocoord(gsum_ref, pdif_ref, ea_ref, w1dr, w1e, b1,
                       w2, b2, m_ref):
    diff = pdif_ref[...]
    d2 = jnp.clip(jnp.sum(diff * diff, axis=-1, keepdims=True), 0.0, 1000.0)
    pre = gsum_ref[...] + d2 * w1dr[...] \
        + _dot(ea_ref[...], w1e[...]) + b1[...]
    m_ref[...] = jax.nn.relu(_dot(jax.nn.relu(pre), w2[...]) + b2[...])


def _tc_edge(gsum, pdif, ea, lp, with_coord):
    w1 = lp["edge1"]["w"]
    w1dr = w1[2 * H:2 * H + 1]
    w1e = w1[2 * H + 1:]
    args = [gsum, pdif, ea,
            w1dr, w1e, lp["edge1"]["b"].reshape(1, H),
            lp["edge2"]["w"], lp["edge2"]["b"].reshape(1, H)]
    in_specs = [_row_spec(_BE, H), _row_spec(_BE, H),
                _row_spec(_BE, DE),
                _full_spec((1, H)), _full_spec((DE, H)), _full_spec((1, H)),
                _full_spec((H, H)), _full_spec((1, H))]
    if with_coord:
        args += [lp["coord1"]["w"], lp["coord1"]["b"].reshape(1, H),
                 lp["coord2"]["w"].T, jnp.full((1, H), lp["coord2"]["b"][0])]
        in_specs += [_full_spec((H, H)), _full_spec((1, H)),
                     _full_spec((1, H)), _full_spec((1, H))]
        return pl.pallas_call(
            _edge_body_coord,
            grid=(E // _BE,),
            in_specs=in_specs,
            out_specs=[_row_spec(_BE, H)] * 2,
            out_shape=[jax.ShapeDtypeStruct((E, H), f32)] * 2,
        )(*args)
    return pl.pallas_call(
        _edge_body_nocoord,
        grid=(E // _BE,),
        in_specs=in_specs,
        out_specs=_row_spec(_BE, H),
        out_shape=jax.ShapeDtypeStruct((E, H), f32),
    )(*args)


def _ln(x, g, b, eps=1e-5):
    mu = jnp.mean(x, axis=-1, keepdims=True)
    xc = x - mu
    var = jnp.mean(xc * xc, axis=-1, keepdims=True)
    return xc / jnp.sqrt(var + eps) * g + b


def _node_body(h_ref, a_ref, dp_ref, pp_ref, wna, wnb, bn, g, b, w1s, w1d,
               ho_ref, ts_ref, td_ref, ppo_ref, pno_ref):
    h = h_ref[...]
    hu = jax.nn.relu(_dot(h, wna[...]) + _dot(a_ref[...], wnb[...]) + bn[...])
    hn = _ln(h + hu, g[...], b[...])
    ho_ref[...] = hn
    ts_ref[...] = _dot(hn, w1s[...])
    td_ref[...] = _dot(hn, w1d[...])
    ppo = pp_ref[...] + dp_ref[...]
    ppo_ref[...] = ppo
    pno_ref[...] = -ppo


def _tc_node(h, a, dp, pp, lp, w1s_next, w1d_next):
    wn = lp["node1"]["w"]
    args = (h, a, dp, pp,
            wn[:H], wn[H:], lp["node1"]["b"].reshape(1, H),
            lp["node_norm"]["g"].reshape(1, H),
            lp["node_norm"]["b"].reshape(1, H),
            w1s_next, w1d_next)
    in_specs = [_row_spec(_BN, H)] * 4 + [
        _full_spec((H, H)), _full_spec((H, H)), _full_spec((1, H)),
        _full_spec((1, H)), _full_spec((1, H)),
        _full_spec((H, H)), _full_spec((H, H))]
    return pl.pallas_call(
        _node_body,
        grid=(N // _BN,),
        in_specs=in_specs,
        out_specs=[_row_spec(_BN, H)] * 5,
        out_shape=[jax.ShapeDtypeStruct((N, H), f32)] * 5,
    )(*args)


def _final_body(h_ref, a0_ref, a1_ref, wna, wnb, bn, g, b, gf, bf, wh, bh,
                out_ref):
    h = h_ref[...]
    agg = a0_ref[...] + a1_ref[...]
    hu = jax.nn.relu(_dot(h, wna[...]) + _dot(agg, wnb[...]) + bn[...])
    hn = _ln(h + hu, g[...], b[...])
    hf = _ln(hn, gf[...], bf[...])
    out_ref[...] = _dot(hf, wh[...]) + bh[...]


def _tc_final(h, a0, a1, lp, p):
    wn = lp["node1"]["w"]
    wh = jnp.pad(p["head"]["w"], ((0, 0), (0, H - 20)))
    bh = jnp.pad(p["head"]["b"], (0, H - 20)).reshape(1, H)
    args = (h, a0, a1,
            wn[:H], wn[H:], lp["node1"]["b"].reshape(1, H),
            lp["node_norm"]["g"].reshape(1, H),
            lp["node_norm"]["b"].reshape(1, H),
            p["final_norm"]["g"].reshape(1, H),
            p["final_norm"]["b"].reshape(1, H),
            wh, bh)
    in_specs = [_row_spec(_BN, H)] * 3 + [
        _full_spec((H, H)), _full_spec((H, H)), _full_spec((1, H)),
        _full_spec((1, H)), _full_spec((1, H)),
        _full_spec((1, H)), _full_spec((1, H)),
        _full_spec((H, H)), _full_spec((1, H)),
    ]
    return pl.pallas_call(
        _final_body,
        grid=(N // _BN,),
        in_specs=in_specs,
        out_specs=_row_spec(_BN, H),
        out_shape=jax.ShapeDtypeStruct((N, H), f32),
    )(*args)


def _w1_parts(lp):
    w1 = lp["edge1"]["w"]
    return w1[:H], w1[H:2 * H]


def kernel(x_struct, x_esm, edge_index, edge_attr, pos, params):
    src = edge_index[0]
    dst = edge_index[1]
    pp = jnp.pad(pos, ((0, 0), (0, H - 3)))
    pn = -pp
    zeros_n = jnp.zeros((N, H), f32)
    layers = params["layers"]

    w1s0, w1d0 = _w1_parts(layers[0])
    h, ts, td = _tc_fuse(x_struct, x_esm, params, w1s0, w1d0)

    for l in range(4):
        lp = layers[l]
        gsum, pdif = _sc_gather(ts, td, pp, pn, src, dst)
        if l < 3:
            m, tr = _tc_edge(gsum, pdif, edge_attr, lp, True)
            a, dp = _sc_scatter2(m, tr, dst, zeros_n)
            w1s_n, w1d_n = _w1_parts(layers[l + 1])
            h, ts, td, pp, pn = _tc_node(h, a, dp, pp, lp, w1s_n, w1d_n)
        else:
            m = _tc_edge(gsum, pdif, edge_attr, lp, False)
            a0, a1 = _sc_scatter1(m, dst, zeros_n)
            out = _tc_final(h, a0, a1, lp, params)

    return out[:, :20]
